# baseline (speedup 1.0000x reference)
import functools

import jax
import jax.numpy as jnp
from jax import lax
from jax.experimental import pallas as pl
from jax.experimental.pallas import tpu as pltpu
from jax.experimental.pallas import tpu_sc as plsc

B = 16384
E = 16
NTAB = 1000000
NC = 2
NS = 16
NW = NC * NS
N1 = B // NW
CH = 128
NQ = N1 // CH
NST = E * NQ
NEL = N1 * E

_mesh = plsc.VectorSubcoreMesh(core_axis_name="c", subcore_axis_name="s",
                               num_cores=NC)


@functools.partial(
    pl.kernel,
    mesh=_mesh,
    compiler_params=pltpu.CompilerParams(use_tc_tiling_on_sc=False),
    out_type=(
        jax.ShapeDtypeStruct((NW * 16,), jnp.float32),
        jax.ShapeDtypeStruct((B,), jnp.float32),
    ),
    scratch_types=[
        pltpu.VMEM((N1,), jnp.int32),
        pltpu.VMEM((N1,), jnp.int32),
        pltpu.VMEM((NST, CH), jnp.int32),
        pltpu.VMEM((NST, CH), jnp.int32),
        pltpu.VMEM((NEL,), jnp.float32),
        pltpu.VMEM((NEL,), jnp.float32),
        pltpu.VMEM((N1,), jnp.float32),
        pltpu.VMEM((N1,), jnp.float32),
        pltpu.VMEM((N1,), jnp.float32),
        pltpu.VMEM((16,), jnp.float32),
        pltpu.SemaphoreType.DMA,
    ],
)
def _sc_phase1(uidx_hbm, iidx_hbm, ut_hbm, ub_hbm, it_hbm, ib_hbm,
               part_hbm, bsum_hbm,
               uidx_v, iidx_v, idx_u, idx_i, u_g, i_g, ub_g, ib_g,
               bsum_v, acc_v, sem):
    wid = lax.axis_index("s") * NC + lax.axis_index("c")
    base = wid * N1

    pltpu.sync_copy(uidx_hbm.at[pl.ds(base, N1)], uidx_v)
    pltpu.sync_copy(iidx_hbm.at[pl.ds(base, N1)], iidx_v)

    def build_body(j, carry):
        e = j // NQ
        q = j % NQ
        for g in range(CH // 16):
            vu = uidx_v[pl.ds(q * CH + g * 16, 16)] + e * NTAB
            vi = iidx_v[pl.ds(q * CH + g * 16, 16)] + e * NTAB
            idx_u[j, pl.ds(g * 16, 16)] = vu
            idx_i[j, pl.ds(g * 16, 16)] = vi
        return carry

    lax.fori_loop(0, NST, build_body, 0)

    def fire_body(j, carry):
        pltpu.async_copy(ut_hbm.at[idx_u.at[j]],
                         u_g.at[pl.ds(j * CH, CH)], sem)
        pltpu.async_copy(it_hbm.at[idx_i.at[j]],
                         i_g.at[pl.ds(j * CH, CH)], sem)
        return carry

    lax.fori_loop(0, NST, fire_body, 0)
    for q in range(NQ):
        pltpu.async_copy(ub_hbm.at[idx_u.at[q]],
                         ub_g.at[pl.ds(q * CH, CH)], sem)
        pltpu.async_copy(ib_hbm.at[idx_i.at[q]],
                         ib_g.at[pl.ds(q * CH, CH)], sem)
    pltpu.make_async_copy(ut_hbm.at[pl.ds(0, NEL)], u_g, sem).wait()
    pltpu.make_async_copy(it_hbm.at[pl.ds(0, NEL)], i_g, sem).wait()
    pltpu.make_async_copy(ub_hbm.at[pl.ds(0, N1)], ub_g, sem).wait()
    pltpu.make_async_copy(ib_hbm.at[pl.ds(0, N1)], ib_g, sem).wait()

    zero = jnp.zeros((16,), jnp.float32)

    def dot_body(i, accs):
        a0, a1, a2, a3 = accs
        r = i * 64
        a0 = a0 + u_g[pl.ds(r, 16)] * i_g[pl.ds(r, 16)]
        a1 = a1 + u_g[pl.ds(r + 16, 16)] * i_g[pl.ds(r + 16, 16)]
        a2 = a2 + u_g[pl.ds(r + 32, 16)] * i_g[pl.ds(r + 32, 16)]
        a3 = a3 + u_g[pl.ds(r + 48, 16)] * i_g[pl.ds(r + 48, 16)]
        return (a0, a1, a2, a3)

    a0, a1, a2, a3 = lax.fori_loop(0, NEL // 64, dot_body,
                                   (zero, zero, zero, zero))
    acc_v[...] = (a0 + a1) + (a2 + a3)
    pltpu.sync_copy(acc_v, part_hbm.at[pl.ds(wid * 16, 16)])

    def bs_body(i, carry):
        sl = pl.ds(i * 16, 16)
        bsum_v[sl] = ub_g[sl] + ib_g[sl]
        return carry

    lax.fori_loop(0, N1 // 16, bs_body, 0)
    pltpu.sync_copy(bsum_v, bsum_hbm.at[pl.ds(base, N1)])


@functools.partial(
    pl.kernel,
    mesh=_mesh,
    compiler_params=pltpu.CompilerParams(use_tc_tiling_on_sc=False),
    out_type=jax.ShapeDtypeStruct((B,), jnp.float32),
    scratch_types=[
        pltpu.VMEM((NW * 16,), jnp.float32),
        pltpu.VMEM((N1,), jnp.float32),
        pltpu.VMEM((N1,), jnp.float32),
    ],
)
def _sc_phase2(part_hbm, bsum_hbm, out_hbm, part_v, bsum_v, out_v):
    wid = lax.axis_index("s") * NC + lax.axis_index("c")
    base = wid * N1

    pltpu.sync_copy(part_hbm, part_v)
    pltpu.sync_copy(bsum_hbm.at[pl.ds(base, N1)], bsum_v)

    tot = jnp.zeros((16,), jnp.float32)
    for w in range(NW):
        tot = tot + part_v[pl.ds(w * 16, 16)]
    dnums = lax.GatherDimensionNumbers(
        offset_dims=(), collapsed_slice_dims=(0,), start_index_map=(0,))
    for sh in (1, 2, 4, 8):
        perm = lax.iota(jnp.int32, 16) ^ sh
        tot = tot + lax.gather(
            tot, perm[:, None], dnums, (1,),
            mode=lax.GatherScatterMode.PROMISE_IN_BOUNDS)

    def out_body(i, carry):
        sl = pl.ds(i * 16, 16)
        out_v[sl] = jnp.maximum(tot + bsum_v[sl], 0.0)
        return carry

    lax.fori_loop(0, N1 // 16, out_body, 0)
    pltpu.sync_copy(out_v, out_hbm.at[pl.ds(base, N1)])


def kernel(inputs, user_table, user_bias_table, item_table, item_bias_table):
    idx32 = inputs.astype(jnp.int32)
    user_idx = jnp.sum(idx32 * jnp.array([1, 0], jnp.int32), axis=1)
    item_idx = jnp.sum(idx32 * jnp.array([0, 1], jnp.int32), axis=1)
    ut = user_table.T.reshape(-1)
    it = item_table.T.reshape(-1)
    ub = user_bias_table.T.reshape(-1)
    ib = item_bias_table.T.reshape(-1)
    part, bsum = _sc_phase1(user_idx, item_idx, ut, ub, it, ib)
    out = _sc_phase2(part, bsum)
    return out.reshape(B, 1)


# speedup vs baseline: 1.7132x; 1.7132x over previous
import functools

import jax
import jax.numpy as jnp
from jax import lax
from jax.experimental import pallas as pl
from jax.experimental.pallas import tpu as pltpu
from jax.experimental.pallas import tpu_sc as plsc

B = 16384
E = 16
NTAB = 1000000
NC = 2
NS = 16
NW = NC * NS
N1 = B // NW
CH = 128
NQ = N1 // CH
TBLK = 2048

_mesh = plsc.VectorSubcoreMesh(core_axis_name="c", subcore_axis_name="s",
                               num_cores=NC)


def _transpose_body(in_ref, out_ref):
    out_ref[...] = in_ref[...].T


_tc_transpose = pl.pallas_call(
    _transpose_body,
    grid=(pl.cdiv(NTAB, TBLK),),
    in_specs=[pl.BlockSpec((E, TBLK), lambda i: (0, i))],
    out_specs=pl.BlockSpec((TBLK, E), lambda i: (i, 0)),
    out_shape=jax.ShapeDtypeStruct((NTAB, E), jnp.float32),
)


@functools.partial(
    pl.kernel,
    mesh=_mesh,
    compiler_params=pltpu.CompilerParams(use_tc_tiling_on_sc=False),
    out_type=(
        jax.ShapeDtypeStruct((NW * 16,), jnp.float32),
        jax.ShapeDtypeStruct((B,), jnp.float32),
    ),
    scratch_types=[
        pltpu.VMEM((NQ, CH), jnp.int32),
        pltpu.VMEM((NQ, CH), jnp.int32),
        pltpu.VMEM((N1, E), jnp.float32),
        pltpu.VMEM((N1, E), jnp.float32),
        pltpu.VMEM((N1,), jnp.float32),
        pltpu.VMEM((N1,), jnp.float32),
        pltpu.VMEM((N1,), jnp.float32),
        pltpu.VMEM((16,), jnp.float32),
        pltpu.SemaphoreType.DMA,
    ],
)
def _sc_phase1(uidx_hbm, iidx_hbm, ut_hbm, ub_hbm, it_hbm, ib_hbm,
               part_hbm, bsum_hbm,
               uidx_v, iidx_v, urows_v, irows_v, ub_g, ib_g,
               bsum_v, acc_v, sem):
    wid = lax.axis_index("s") * NC + lax.axis_index("c")
    base_ch = wid * NQ

    pltpu.sync_copy(uidx_hbm.at[pl.ds(base_ch, NQ)], uidx_v)
    pltpu.sync_copy(iidx_hbm.at[pl.ds(base_ch, NQ)], iidx_v)

    for q in range(NQ):
        pltpu.async_copy(ut_hbm.at[uidx_v.at[q]],
                         urows_v.at[pl.ds(q * CH, CH)], sem)
        pltpu.async_copy(it_hbm.at[iidx_v.at[q]],
                         irows_v.at[pl.ds(q * CH, CH)], sem)
        pltpu.async_copy(ub_hbm.at[uidx_v.at[q]],
                         ub_g.at[pl.ds(q * CH, CH)], sem)
        pltpu.async_copy(ib_hbm.at[iidx_v.at[q]],
                         ib_g.at[pl.ds(q * CH, CH)], sem)
    pltpu.make_async_copy(ut_hbm.at[pl.ds(0, N1)], urows_v, sem).wait()
    pltpu.make_async_copy(it_hbm.at[pl.ds(0, N1)], irows_v, sem).wait()
    pltpu.make_async_copy(ub_hbm.at[pl.ds(0, N1)], ub_g, sem).wait()
    pltpu.make_async_copy(ib_hbm.at[pl.ds(0, N1)], ib_g, sem).wait()

    zero = jnp.zeros((E,), jnp.float32)

    def dot_body(i, accs):
        a0, a1, a2, a3 = accs
        r = i * 4
        a0 = a0 + urows_v[r] * irows_v[r]
        a1 = a1 + urows_v[r + 1] * irows_v[r + 1]
        a2 = a2 + urows_v[r + 2] * irows_v[r + 2]
        a3 = a3 + urows_v[r + 3] * irows_v[r + 3]
        return (a0, a1, a2, a3)

    a0, a1, a2, a3 = lax.fori_loop(0, N1 // 4, dot_body,
                                   (zero, zero, zero, zero))
    acc_v[...] = (a0 + a1) + (a2 + a3)
    pltpu.sync_copy(acc_v, part_hbm.at[pl.ds(wid * 16, 16)])

    def bs_body(i, carry):
        sl = pl.ds(i * 16, 16)
        bsum_v[sl] = ub_g[sl] + ib_g[sl]
        return carry

    lax.fori_loop(0, N1 // 16, bs_body, 0)
    pltpu.sync_copy(bsum_v, bsum_hbm.at[pl.ds(wid * N1, N1)])


@functools.partial(
    pl.kernel,
    mesh=_mesh,
    compiler_params=pltpu.CompilerParams(use_tc_tiling_on_sc=False),
    out_type=jax.ShapeDtypeStruct((B,), jnp.float32),
    scratch_types=[
        pltpu.VMEM((NW * 16,), jnp.float32),
        pltpu.VMEM((N1,), jnp.float32),
        pltpu.VMEM((N1,), jnp.float32),
    ],
)
def _sc_phase2(part_hbm, bsum_hbm, out_hbm, part_v, bsum_v, out_v):
    wid = lax.axis_index("s") * NC + lax.axis_index("c")
    base = wid * N1

    pltpu.sync_copy(part_hbm, part_v)
    pltpu.sync_copy(bsum_hbm.at[pl.ds(base, N1)], bsum_v)

    tot = jnp.zeros((16,), jnp.float32)
    for w in range(NW):
        tot = tot + part_v[pl.ds(w * 16, 16)]
    dnums = lax.GatherDimensionNumbers(
        offset_dims=(), collapsed_slice_dims=(0,), start_index_map=(0,))
    for sh in (1, 2, 4, 8):
        perm = lax.iota(jnp.int32, 16) ^ sh
        tot = tot + lax.gather(
            tot, perm[:, None], dnums, (1,),
            mode=lax.GatherScatterMode.PROMISE_IN_BOUNDS)

    def out_body(i, carry):
        sl = pl.ds(i * 16, 16)
        out_v[sl] = jnp.maximum(tot + bsum_v[sl], 0.0)
        return carry

    lax.fori_loop(0, N1 // 16, out_body, 0)
    pltpu.sync_copy(out_v, out_hbm.at[pl.ds(base, N1)])


def kernel(inputs, user_table, user_bias_table, item_table, item_bias_table):
    idx32 = inputs.astype(jnp.int32)
    user_idx = jnp.sum(idx32 * jnp.array([1, 0], jnp.int32), axis=1)
    item_idx = jnp.sum(idx32 * jnp.array([0, 1], jnp.int32), axis=1)
    uidx2 = user_idx.reshape(NW * NQ, CH)
    iidx2 = item_idx.reshape(NW * NQ, CH)
    utr = _tc_transpose(user_table.T)
    itr = _tc_transpose(item_table.T)
    ub = user_bias_table.reshape(-1)
    ib = item_bias_table.reshape(-1)
    part, bsum = _sc_phase1(uidx2, iidx2, utr, ub, itr, ib)
    out = _sc_phase2(part, bsum)
    return out.reshape(B, 1)


# speedup vs baseline: 4.2905x; 2.5044x over previous
import functools

import jax
import jax.numpy as jnp
from jax import lax
from jax.experimental import pallas as pl
from jax.experimental.pallas import tpu as pltpu
from jax.experimental.pallas import tpu_sc as plsc

B = 16384
E = 16
NC = 2
NS = 16
NW = NC * NS
N1 = B // NW
CH = 128
NWAVES = N1 // CH
NQ = N1 // 128

_mesh = plsc.VectorSubcoreMesh(core_axis_name="c", subcore_axis_name="s",
                               num_cores=NC)


WAVE = 128
WT = WAVE // 8


@functools.partial(
    pl.kernel,
    mesh=_mesh,
    out_type=(
        jax.ShapeDtypeStruct((NW * 16,), jnp.float32),
        jax.ShapeDtypeStruct((WT, 8, E), jnp.float32),
    ),
    scratch_types=[
        pltpu.VMEM((N1,), jnp.int32),
        pltpu.VMEM((N1,), jnp.int32),
        pltpu.VMEM((WT, 8, E), jnp.float32),
        pltpu.VMEM((WT, 8, E), jnp.float32),
        pltpu.VMEM((16,), jnp.float32),
        pltpu.SemaphoreType.DMA,
    ],
)
def _sc_phase1(uidx_hbm, iidx_hbm, ut_hbm, it_hbm,
               part_hbm, dummy_hbm,
               uidx_v, iidx_v, u_t, i_t, acc_v, sem):
    wid = lax.axis_index("s") * NC + lax.axis_index("c")
    base = wid * N1

    pltpu.sync_copy(uidx_hbm.at[pl.ds(base, N1)], uidx_v)
    pltpu.sync_copy(iidx_hbm.at[pl.ds(base, N1)], iidx_v)

    zero = jnp.zeros((E,), jnp.float32)

    def wave_body(w, accs):
        for j in range(WAVE // 16):
            su = uidx_v[pl.ds(w * WAVE + j * 16, 16)]
            si = iidx_v[pl.ds(w * WAVE + j * 16, 16)]
            for k in range(16):
                r = j * 16 + k
                pltpu.async_copy(ut_hbm.at[pl.ds(su[k], 1)],
                                 u_t.at[r // 8, pl.ds(r % 8, 1), :], sem)
                pltpu.async_copy(it_hbm.at[pl.ds(si[k], 1)],
                                 i_t.at[r // 8, pl.ds(r % 8, 1), :], sem)
        pltpu.make_async_copy(dummy_hbm, u_t, sem).wait()
        pltpu.make_async_copy(dummy_hbm, i_t, sem).wait()

        a0, a1, a2, a3 = accs
        for j in range(WAVE // 8):
            a0 = a0 + u_t[j, 0, :] * i_t[j, 0, :]
            a1 = a1 + u_t[j, 1, :] * i_t[j, 1, :]
            a2 = a2 + u_t[j, 2, :] * i_t[j, 2, :]
            a3 = a3 + u_t[j, 3, :] * i_t[j, 3, :]
            a0 = a0 + u_t[j, 4, :] * i_t[j, 4, :]
            a1 = a1 + u_t[j, 5, :] * i_t[j, 5, :]
            a2 = a2 + u_t[j, 6, :] * i_t[j, 6, :]
            a3 = a3 + u_t[j, 7, :] * i_t[j, 7, :]
        return (a0, a1, a2, a3)

    a0, a1, a2, a3 = lax.fori_loop(0, N1 // WAVE, wave_body,
                                   (zero, zero, zero, zero))
    acc_v[...] = (a0 + a1) + (a2 + a3)
    pltpu.sync_copy(acc_v, part_hbm.at[pl.ds(wid * 16, 16)])


@functools.partial(
    pl.kernel,
    mesh=_mesh,
    compiler_params=pltpu.CompilerParams(use_tc_tiling_on_sc=False),
    out_type=jax.ShapeDtypeStruct((B,), jnp.float32),
    scratch_types=[
        pltpu.VMEM((NQ, 128), jnp.int32),
        pltpu.VMEM((NQ, 128), jnp.int32),
        pltpu.VMEM((NW * 16,), jnp.float32),
        pltpu.VMEM((N1,), jnp.float32),
        pltpu.VMEM((N1,), jnp.float32),
        pltpu.VMEM((N1,), jnp.float32),
        pltpu.SemaphoreType.DMA,
    ],
)
def _sc_phase2(uidx_hbm, iidx_hbm, ub_hbm, ib_hbm, part_hbm, out_hbm,
               uidx_v, iidx_v, part_v, ub_g, ib_g, out_v, sem):
    wid = lax.axis_index("s") * NC + lax.axis_index("c")
    base = wid * N1
    base_ch = wid * NQ

    pltpu.sync_copy(uidx_hbm.at[pl.ds(base_ch, NQ)], uidx_v)
    pltpu.sync_copy(iidx_hbm.at[pl.ds(base_ch, NQ)], iidx_v)
    for q in range(NQ):
        pltpu.async_copy(ub_hbm.at[uidx_v.at[q]],
                         ub_g.at[pl.ds(q * 128, 128)], sem)
        pltpu.async_copy(ib_hbm.at[iidx_v.at[q]],
                         ib_g.at[pl.ds(q * 128, 128)], sem)
    pltpu.sync_copy(part_hbm, part_v)
    pltpu.make_async_copy(ub_hbm.at[pl.ds(0, N1)], ub_g, sem).wait()
    pltpu.make_async_copy(ib_hbm.at[pl.ds(0, N1)], ib_g, sem).wait()

    tot = jnp.zeros((16,), jnp.float32)
    for w in range(NW):
        tot = tot + part_v[pl.ds(w * 16, 16)]
    dnums = lax.GatherDimensionNumbers(
        offset_dims=(), collapsed_slice_dims=(0,), start_index_map=(0,))
    for sh in (1, 2, 4, 8):
        perm = lax.iota(jnp.int32, 16) ^ sh
        tot = tot + lax.gather(
            tot, perm[:, None], dnums, (1,),
            mode=lax.GatherScatterMode.PROMISE_IN_BOUNDS)

    def out_body(i, carry):
        sl = pl.ds(i * 16, 16)
        out_v[sl] = jnp.maximum(tot + ub_g[sl] + ib_g[sl], 0.0)
        return carry

    lax.fori_loop(0, N1 // 16, out_body, 0)
    pltpu.sync_copy(out_v, out_hbm.at[pl.ds(base, N1)])


def kernel(inputs, user_table, user_bias_table, item_table, item_bias_table):
    idx32 = inputs.astype(jnp.int32)
    user_idx = jnp.sum(idx32 * jnp.array([1, 0], jnp.int32), axis=1)
    item_idx = jnp.sum(idx32 * jnp.array([0, 1], jnp.int32), axis=1)
    uidx2 = user_idx.reshape(NW * NQ, 128)
    iidx2 = item_idx.reshape(NW * NQ, 128)
    ub = user_bias_table.reshape(-1)
    ib = item_bias_table.reshape(-1)
    part, _ = _sc_phase1(user_idx, item_idx, user_table, item_table)
    out = _sc_phase2(uidx2, iidx2, ub, ib, part)
    return out.reshape(B, 1)
